# baseline (speedup 1.0000x reference)
import functools

import jax
import jax.numpy as jnp
from jax import lax
from jax.experimental import pallas as pl
from jax.experimental.pallas import tpu as pltpu
from jax.experimental.pallas import tpu_sc as plsc

DIM = 64


def _transh_sc(h_ids, r_ids, t_ids, ent_flat, relation_emb, normal_vec):
    B = h_ids.shape[0]
    NE = ent_flat.shape[0] // DIM
    NC, NS, L = 2, 16, 16
    NW = NC * NS
    PW = B // NW
    C = min(256, PW)
    NCH = PW // C
    IH = min(128, C)
    G = C // L
    EPC = C * DIM

    mesh = plsc.VectorSubcoreMesh(
        core_axis_name="c", subcore_axis_name="s", num_cores=NC, num_subcores=NS)

    @functools.partial(
        pl.kernel,
        mesh=mesh,
        out_type=jax.ShapeDtypeStruct((B,), jnp.float32),
        compiler_params=pltpu.CompilerParams(
            needs_layout_passes=False, use_tc_tiling_on_sc=False),
        scratch_types=[
            pltpu.VMEM((C,), jnp.int32),
            pltpu.VMEM((C,), jnp.int32),
            [pltpu.VMEM((IH,), jnp.int32) for _ in range(C // IH)],
            pltpu.VMEM((EPC,), jnp.int32),
            pltpu.VMEM((EPC,), jnp.int32),
            pltpu.VMEM((EPC,), jnp.float32),
            pltpu.VMEM((EPC,), jnp.float32),
            pltpu.VMEM((C, DIM), jnp.float32),
            pltpu.VMEM((C, DIM), jnp.float32),
            pltpu.VMEM((DIM, L), jnp.float32),
            pltpu.VMEM((PW,), jnp.float32),
            pltpu.SemaphoreType.DMA,
        ],
    )
    def _k(h_hbm, r_hbm, t_hbm, ent_hbm, rel_hbm, nrm_hbm, out_hbm,
           hids, tids, ridx, hibuf, tibuf, hdat, tdat, rrows, nrows,
           u_scr, outv, sem):
        wid = lax.axis_index("s") * NC + lax.axis_index("c")
        lane = lax.iota(jnp.int32, L)

        for ch in range(NCH):
            base = wid * PW + ch * C
            pltpu.sync_copy(h_hbm.at[pl.ds(base, C)], hids)
            pltpu.sync_copy(t_hbm.at[pl.ds(base, C)], tids)
            for i in range(C // IH):
                pltpu.sync_copy(r_hbm.at[pl.ds(base + i * IH, IH)], ridx[i])

            @pl.loop(0, G)
            def _build(g):
                hv = hids[pl.ds(g * L, L)]
                tv = tids[pl.ds(g * L, L)]
                for d in range(DIM):
                    off = pl.ds(g * DIM * L + d * L, L)
                    hibuf[off] = hv
                    tibuf[off] = tv
                    if d + 1 < DIM:
                        hv = hv + NE
                        tv = tv + NE

            @pl.loop(0, EPC // IH)
            def _fire(p):
                sl = pl.ds(p * IH, IH)
                pltpu.async_copy(ent_hbm.at[hibuf.at[sl]], hdat.at[sl], sem)
                pltpu.async_copy(ent_hbm.at[tibuf.at[sl]], tdat.at[sl], sem)

            for i in range(C // IH):
                s = i * IH
                pltpu.async_copy(
                    rel_hbm.at[ridx[i]], rrows.at[pl.ds(s, IH)], sem)
                pltpu.async_copy(
                    nrm_hbm.at[ridx[i]], nrows.at[pl.ds(s, IH)], sem)
            pltpu.make_async_copy(ent_hbm.at[pl.ds(0, EPC)], hdat, sem).wait()
            pltpu.make_async_copy(ent_hbm.at[pl.ds(0, EPC)], tdat, sem).wait()
            pltpu.make_async_copy(rel_hbm.at[pl.ds(0, C)], rrows, sem).wait()
            pltpu.make_async_copy(nrm_hbm.at[pl.ds(0, C)], nrows, sem).wait()

            @pl.loop(0, G)
            def _group(g):
                row = g * L + lane
                cd = jnp.zeros((L,), jnp.int32)
                un = jnp.zeros((L,), jnp.float32)
                nn = jnp.zeros((L,), jnp.float32)
                for d in range(DIM):
                    off = pl.ds(g * DIM * L + d * L, L)
                    hv = hdat[off]
                    tv = tdat[off]
                    nv = plsc.load_gather(nrows, [row, cd])
                    uv = hv - tv
                    u_scr[d] = uv
                    un = un + uv * nv
                    nn = nn + nv * nv
                    if d + 1 < DIM:
                        cd = cd + 1
                gamma = un / jnp.maximum(nn, 1e-24)
                cd2 = jnp.zeros((L,), jnp.int32)
                acc = jnp.zeros((L,), jnp.float32)
                for d in range(DIM):
                    rv = plsc.load_gather(rrows, [row, cd2])
                    nv = plsc.load_gather(nrows, [row, cd2])
                    acc = acc + jnp.abs(u_scr[d] + rv - gamma * nv)
                    if d + 1 < DIM:
                        cd2 = cd2 + 1
                outv[pl.ds(ch * C + g * L, L)] = acc

        pltpu.sync_copy(outv, out_hbm.at[pl.ds(wid * PW, PW)])

    return _k(h_ids, r_ids, t_ids, ent_flat, relation_emb, normal_vec)


def kernel(h_ids, r_ids, t_ids, entity_emb, relation_emb, normal_vec):
    ent_flat = entity_emb.T.reshape(-1)
    return _transh_sc(h_ids, r_ids, t_ids, ent_flat, relation_emb, normal_vec)


# speedup vs baseline: 9.2462x; 9.2462x over previous
import functools

import jax
import jax.numpy as jnp
from jax import lax
from jax.experimental import pallas as pl
from jax.experimental.pallas import tpu as pltpu
from jax.experimental.pallas import tpu_sc as plsc

DIM = 64


def _transh_sc(h_ids, r_ids, t_ids, entity_emb, rn_table):
    B = h_ids.shape[0]
    NC, NS, L = 2, 16, 16
    NW = NC * NS
    PW = B // NW
    C = min(128, PW)
    NCH = PW // C
    G = C // L
    KR = 16

    mesh = plsc.VectorSubcoreMesh(
        core_axis_name="c", subcore_axis_name="s", num_cores=NC, num_subcores=NS)

    @functools.partial(
        pl.kernel,
        mesh=mesh,
        out_type=jax.ShapeDtypeStruct((B,), jnp.float32),
        compiler_params=pltpu.CompilerParams(
            needs_layout_passes=False, use_tc_tiling_on_sc=True),
        scratch_types=[
            pltpu.VMEM((C + L,), jnp.int32),
            pltpu.VMEM((C + L,), jnp.int32),
            pltpu.VMEM((C,), jnp.int32),
            pltpu.VMEM((C, 2 * DIM), jnp.float32),
            pltpu.VMEM((DIM, C), jnp.float32),
            pltpu.VMEM((DIM, C), jnp.float32),
            pltpu.VMEM((KR, 8, DIM), jnp.float32),
            pltpu.VMEM((DIM, L), jnp.float32),
            pltpu.VMEM((PW,), jnp.float32),
            pltpu.SemaphoreType.DMA,
            pltpu.SemaphoreType.DMA,
        ],
    )
    def _k(h_hbm, r_hbm, t_hbm, ent_hbm, rn_hbm, out_hbm,
           hids, tids, rids, rn_rows, hcol, tcol, ring, u_scr, outv,
           sem, sem_rn):
        wid = lax.axis_index("s") * NC + lax.axis_index("c")
        lane = lax.iota(jnp.int32, L)

        def fetch(j, idsbuf):
            e = idsbuf[pl.ds(j, L)][0]
            blk = pl.multiple_of((e >> 3) * 8, 8)
            p = j & (KR - 1)
            pltpu.async_copy(
                ent_hbm.at[pl.ds(blk, 8), :], ring.at[p], sem)

        def extract(j, idsbuf, dstcol):
            e = idsbuf[pl.ds(j, L)][0]
            sub = e & 7
            p = j & (KR - 1)
            for k in range(DIM // L):
                vals = ring[p, sub, pl.ds(k * L, L)]
                plsc.store_scatter(
                    dstcol, [k * L + lane, jnp.full((L,), j, jnp.int32)], vals)

        @pl.loop(0, NCH)
        def _chunk(ch):
            base = wid * PW + ch * C
            pltpu.sync_copy(h_hbm.at[pl.ds(base, C)], hids.at[pl.ds(0, C)])
            pltpu.sync_copy(t_hbm.at[pl.ds(base, C)], tids.at[pl.ds(0, C)])
            pltpu.sync_copy(r_hbm.at[pl.ds(base, C)], rids)
            pltpu.async_copy(rn_hbm.at[rids], rn_rows, sem_rn)

            for idsbuf, dstcol in ((hids, hcol), (tids, tcol)):
                @pl.loop(0, C // KR)
                def _batch(b, idsbuf=idsbuf, dstcol=dstcol):
                    for i in range(KR):
                        fetch(b * KR + i, idsbuf)
                    for i in range(KR):
                        pltpu.make_async_copy(
                            ent_hbm.at[pl.ds(0, 8), :], ring.at[i], sem).wait()
                    for i in range(KR):
                        extract(b * KR + i, idsbuf, dstcol)

            pltpu.make_async_copy(
                rn_hbm.at[pl.ds(0, C)], rn_rows, sem_rn).wait()

            @pl.loop(0, G)
            def _group(g):
                row = g * L + lane
                cdn = jnp.full((L,), DIM, jnp.int32)
                un = jnp.zeros((L,), jnp.float32)
                nn = jnp.zeros((L,), jnp.float32)
                for d in range(DIM):
                    hv = hcol[d, pl.ds(g * L, L)]
                    tv = tcol[d, pl.ds(g * L, L)]
                    nv = plsc.load_gather(rn_rows, [row, cdn])
                    uv = hv - tv
                    u_scr[d] = uv
                    un = un + uv * nv
                    nn = nn + nv * nv
                    if d + 1 < DIM:
                        cdn = cdn + 1
                gamma = un / jnp.maximum(nn, 1e-24)
                cdr = jnp.zeros((L,), jnp.int32)
                cdn2 = jnp.full((L,), DIM, jnp.int32)
                acc = jnp.zeros((L,), jnp.float32)
                for d in range(DIM):
                    rv = plsc.load_gather(rn_rows, [row, cdr])
                    nv = plsc.load_gather(rn_rows, [row, cdn2])
                    acc = acc + jnp.abs(u_scr[d] + rv - gamma * nv)
                    if d + 1 < DIM:
                        cdr = cdr + 1
                        cdn2 = cdn2 + 1
                outv[pl.ds(ch * C + g * L, L)] = acc

        pltpu.sync_copy(outv, out_hbm.at[pl.ds(wid * PW, PW)])

    return _k(h_ids, r_ids, t_ids, entity_emb, rn_table)


def kernel(h_ids, r_ids, t_ids, entity_emb, relation_emb, normal_vec):
    rn_table = jnp.concatenate([relation_emb, normal_vec], axis=1)
    return _transh_sc(h_ids, r_ids, t_ids, entity_emb, rn_table)


# speedup vs baseline: 11.5102x; 1.2449x over previous
import functools

import jax
import jax.numpy as jnp
from jax import lax
from jax.experimental import pallas as pl
from jax.experimental.pallas import tpu as pltpu
from jax.experimental.pallas import tpu_sc as plsc

DIM = 64


def _pair_rows_tc(ent_t):
    NE = ent_t.shape[1]
    BN = 2048
    grid = pl.cdiv(NE, 2 * BN)

    def body(in_ref, out_ref):
        x = in_ref[...]
        out_ref[...] = jnp.concatenate(
            [x[:, :BN].T, x[:, BN:].T], axis=1)

    return pl.pallas_call(
        body,
        grid=(grid,),
        in_specs=[pl.BlockSpec((DIM, 2 * BN), lambda j: (0, j))],
        out_specs=pl.BlockSpec((BN, 2 * DIM), lambda j: (j, 0)),
        out_shape=jax.ShapeDtypeStruct((grid * BN, 2 * DIM), jnp.float32),
    )(ent_t)


def _transh_sc(h_ids, r_ids, t_ids, ent_pair, rn_table):
    B = h_ids.shape[0]
    NC, NS, L = 2, 16, 16
    NW = NC * NS
    PW = B // NW
    C = min(128, PW)
    NCH = PW // C
    G = C // L

    mesh = plsc.VectorSubcoreMesh(
        core_axis_name="c", subcore_axis_name="s", num_cores=NC, num_subcores=NS)

    @functools.partial(
        pl.kernel,
        mesh=mesh,
        out_type=jax.ShapeDtypeStruct((B,), jnp.float32),
        compiler_params=pltpu.CompilerParams(
            needs_layout_passes=False, use_tc_tiling_on_sc=True),
        scratch_types=[
            pltpu.VMEM((C,), jnp.int32),
            pltpu.VMEM((C,), jnp.int32),
            pltpu.VMEM((C,), jnp.int32),
            pltpu.VMEM((C,), jnp.int32),
            pltpu.VMEM((C,), jnp.int32),
            pltpu.VMEM((C, 2 * DIM), jnp.float32),
            pltpu.VMEM((C, 2 * DIM), jnp.float32),
            pltpu.VMEM((C, 2 * DIM), jnp.float32),
            pltpu.VMEM((DIM, L), jnp.float32),
            pltpu.VMEM((PW,), jnp.float32),
            pltpu.SemaphoreType.DMA,
        ],
    )
    def _k(h_hbm, r_hbm, t_hbm, ent_hbm, rn_hbm, out_hbm,
           hids, tids, hrow, trow, rids, hrows, trows, rn_rows,
           u_scr, outv, sem):
        wid = lax.axis_index("s") * NC + lax.axis_index("c")
        lane = lax.iota(jnp.int32, L)

        @pl.loop(0, NCH)
        def _chunk(ch):
            base = wid * PW + ch * C
            pltpu.sync_copy(h_hbm.at[pl.ds(base, C)], hids)
            pltpu.sync_copy(t_hbm.at[pl.ds(base, C)], tids)
            pltpu.sync_copy(r_hbm.at[pl.ds(base, C)], rids)

            @pl.loop(0, G)
            def _halve(i):
                sl = pl.ds(i * L, L)
                hv = hids[sl]
                tv = tids[sl]
                hrow[sl] = ((hv >> 12) << 11) + (hv & 2047)
                trow[sl] = ((tv >> 12) << 11) + (tv & 2047)

            cp_h = pltpu.async_copy(ent_hbm.at[hrow], hrows, sem)
            cp_t = pltpu.async_copy(ent_hbm.at[trow], trows, sem)
            cp_rn = pltpu.async_copy(rn_hbm.at[rids], rn_rows, sem)
            cp_h.wait()
            cp_t.wait()
            cp_rn.wait()

            @pl.loop(0, G)
            def _group(g):
                row = g * L + lane
                sl = pl.ds(g * L, L)
                cdh = ((hids[sl] >> 11) & 1) * DIM
                cdt = ((tids[sl] >> 11) & 1) * DIM
                cdn = jnp.full((L,), DIM, jnp.int32)
                un = jnp.zeros((L,), jnp.float32)
                nn = jnp.zeros((L,), jnp.float32)
                for d in range(DIM):
                    hv = plsc.load_gather(hrows, [row, cdh])
                    tv = plsc.load_gather(trows, [row, cdt])
                    nv = plsc.load_gather(rn_rows, [row, cdn])
                    uv = hv - tv
                    u_scr[d] = uv
                    un = un + uv * nv
                    nn = nn + nv * nv
                    if d + 1 < DIM:
                        cdh = cdh + 1
                        cdt = cdt + 1
                        cdn = cdn + 1
                gamma = un / jnp.maximum(nn, 1e-24)
                cdr = jnp.zeros((L,), jnp.int32)
                cdn2 = jnp.full((L,), DIM, jnp.int32)
                acc = jnp.zeros((L,), jnp.float32)
                for d in range(DIM):
                    rv = plsc.load_gather(rn_rows, [row, cdr])
                    nv = plsc.load_gather(rn_rows, [row, cdn2])
                    acc = acc + jnp.abs(u_scr[d] + rv - gamma * nv)
                    if d + 1 < DIM:
                        cdr = cdr + 1
                        cdn2 = cdn2 + 1
                outv[pl.ds(ch * C + g * L, L)] = acc

        pltpu.sync_copy(outv, out_hbm.at[pl.ds(wid * PW, PW)])

    return _k(h_ids, r_ids, t_ids, ent_pair, rn_table)


def kernel(h_ids, r_ids, t_ids, entity_emb, relation_emb, normal_vec):
    ent_pair = _pair_rows_tc(entity_emb.T)
    rn_table = jnp.concatenate([relation_emb, normal_vec], axis=1)
    return _transh_sc(h_ids, r_ids, t_ids, ent_pair, rn_table)


# speedup vs baseline: 13.8137x; 1.2001x over previous
import functools

import jax
import jax.numpy as jnp
from jax import lax
from jax.experimental import pallas as pl
from jax.experimental.pallas import tpu as pltpu
from jax.experimental.pallas import tpu_sc as plsc

DIM = 64
PAIR_BN = 4096
PAIR_SH = 12


def _pair_rows_tc(ent_t):
    NE = ent_t.shape[1]
    BN = PAIR_BN
    grid = pl.cdiv(NE, 2 * BN)

    def body(in_ref, out_ref):
        x = in_ref[...]
        out_ref[...] = jnp.concatenate(
            [x[:, :BN].T, x[:, BN:].T], axis=1)

    return pl.pallas_call(
        body,
        grid=(grid,),
        in_specs=[pl.BlockSpec((DIM, 2 * BN), lambda j: (0, j))],
        out_specs=pl.BlockSpec((BN, 2 * DIM), lambda j: (j, 0)),
        out_shape=jax.ShapeDtypeStruct((grid * BN, 2 * DIM), jnp.float32),
        compiler_params=pltpu.CompilerParams(
            fuse_transposed_lhs_in_matmul=True),
    )(ent_t)


def _transh_sc(h_ids, r_ids, t_ids, ent_pair, rn_table):
    B = h_ids.shape[0]
    NC, NS, L = 2, 16, 16
    NW = NC * NS
    PW = B // NW
    C = min(128, PW)
    NCH = PW // C
    G = C // L

    mesh = plsc.VectorSubcoreMesh(
        core_axis_name="c", subcore_axis_name="s", num_cores=NC, num_subcores=NS)

    @functools.partial(
        pl.kernel,
        mesh=mesh,
        out_type=jax.ShapeDtypeStruct((B,), jnp.float32),
        compiler_params=pltpu.CompilerParams(
            needs_layout_passes=False, use_tc_tiling_on_sc=True),
        scratch_types=[
            pltpu.VMEM((2, C), jnp.int32),
            pltpu.VMEM((2, C), jnp.int32),
            pltpu.VMEM((2, C), jnp.int32),
            pltpu.VMEM((2, C), jnp.int32),
            pltpu.VMEM((2, C), jnp.int32),
            pltpu.VMEM((2, C, 2 * DIM), jnp.float32),
            pltpu.VMEM((2, C, 2 * DIM), jnp.float32),
            pltpu.VMEM((2, C, 2 * DIM), jnp.float32),
            pltpu.VMEM((DIM, L), jnp.float32),
            pltpu.VMEM((PW,), jnp.float32),
            pltpu.SemaphoreType.DMA,
            pltpu.SemaphoreType.DMA,
        ],
    )
    def _k(h_hbm, r_hbm, t_hbm, ent_hbm, rn_hbm, out_hbm,
           hids, tids, hrow, trow, rids, hrows, trows, rn_rows,
           u_scr, outv, sem0, sem1):
        wid = lax.axis_index("s") * NC + lax.axis_index("c")
        lane = lax.iota(jnp.int32, L)
        sems = (sem0, sem1)

        def fire(ch, s):
            base = wid * PW + ch * C
            pltpu.sync_copy(h_hbm.at[pl.ds(base, C)], hids.at[s])
            pltpu.sync_copy(t_hbm.at[pl.ds(base, C)], tids.at[s])
            pltpu.sync_copy(r_hbm.at[pl.ds(base, C)], rids.at[s])

            @pl.loop(0, G)
            def _halve(i):
                sl = pl.ds(i * L, L)
                hv = hids[s, sl]
                tv = tids[s, sl]
                hrow[s, sl] = ((hv >> (PAIR_SH + 1)) << PAIR_SH) + (hv & (PAIR_BN - 1))
                trow[s, sl] = ((tv >> (PAIR_SH + 1)) << PAIR_SH) + (tv & (PAIR_BN - 1))

            pltpu.async_copy(ent_hbm.at[hrow.at[s]], hrows.at[s], sems[s])
            pltpu.async_copy(ent_hbm.at[trow.at[s]], trows.at[s], sems[s])
            pltpu.async_copy(rn_hbm.at[rids.at[s]], rn_rows.at[s], sems[s])

        def drain(s):
            pltpu.make_async_copy(
                ent_hbm.at[pl.ds(0, C)], hrows.at[s], sems[s]).wait()
            pltpu.make_async_copy(
                ent_hbm.at[pl.ds(0, C)], trows.at[s], sems[s]).wait()
            pltpu.make_async_copy(
                rn_hbm.at[pl.ds(0, C)], rn_rows.at[s], sems[s]).wait()

        fire(0, 0)
        for ch in range(NCH):
            s = ch & 1
            if ch + 1 < NCH:
                fire(ch + 1, (ch + 1) & 1)
            drain(s)

            @pl.loop(0, G)
            def _group(g, ch=ch, s=s):
                row = g * L + lane
                sl = pl.ds(g * L, L)
                cdh = ((hids[s, sl] >> PAIR_SH) & 1) * DIM
                cdt = ((tids[s, sl] >> PAIR_SH) & 1) * DIM
                cdn = jnp.full((L,), DIM, jnp.int32)
                un = jnp.zeros((L,), jnp.float32)
                nn = jnp.zeros((L,), jnp.float32)
                for d in range(DIM):
                    hv = plsc.load_gather(hrows.at[s], [row, cdh])
                    tv = plsc.load_gather(trows.at[s], [row, cdt])
                    nv = plsc.load_gather(rn_rows.at[s], [row, cdn])
                    uv = hv - tv
                    u_scr[d] = uv
                    un = un + uv * nv
                    nn = nn + nv * nv
                    if d + 1 < DIM:
                        cdh = cdh + 1
                        cdt = cdt + 1
                        cdn = cdn + 1
                gamma = un / jnp.maximum(nn, 1e-24)
                cdr = jnp.zeros((L,), jnp.int32)
                cdn2 = jnp.full((L,), DIM, jnp.int32)
                acc = jnp.zeros((L,), jnp.float32)
                for d in range(DIM):
                    rv = plsc.load_gather(rn_rows.at[s], [row, cdr])
                    nv = plsc.load_gather(rn_rows.at[s], [row, cdn2])
                    acc = acc + jnp.abs(u_scr[d] + rv - gamma * nv)
                    if d + 1 < DIM:
                        cdr = cdr + 1
                        cdn2 = cdn2 + 1
                outv[pl.ds(ch * C + g * L, L)] = acc

        pltpu.sync_copy(outv, out_hbm.at[pl.ds(wid * PW, PW)])

    return _k(h_ids, r_ids, t_ids, ent_pair, rn_table)


def kernel(h_ids, r_ids, t_ids, entity_emb, relation_emb, normal_vec):
    ent_pair = _pair_rows_tc(entity_emb.T)
    rn_table = jnp.concatenate([relation_emb, normal_vec], axis=1)
    return _transh_sc(h_ids, r_ids, t_ids, ent_pair, rn_table)


# speedup vs baseline: 15.1235x; 1.0948x over previous
import functools

import jax
import jax.numpy as jnp
from jax import lax
from jax.experimental import pallas as pl
from jax.experimental.pallas import tpu as pltpu
from jax.experimental.pallas import tpu_sc as plsc

DIM = 64
PAIR_BN = 8192
PAIR_SH = 13


def _pair_rows_tc(ent_t):
    NE = ent_t.shape[1]
    BN = PAIR_BN
    grid = pl.cdiv(NE, 2 * BN)

    def body(in_ref, out_ref):
        x = in_ref[...]
        out_ref[...] = jnp.concatenate(
            [x[:, :BN].T, x[:, BN:].T], axis=1)

    return pl.pallas_call(
        body,
        grid=(grid,),
        in_specs=[pl.BlockSpec((DIM, 2 * BN), lambda j: (0, j))],
        out_specs=pl.BlockSpec((BN, 2 * DIM), lambda j: (j, 0)),
        out_shape=jax.ShapeDtypeStruct((grid * BN, 2 * DIM), jnp.float32),
        compiler_params=pltpu.CompilerParams(
            fuse_transposed_lhs_in_matmul=True),
    )(ent_t)


def _transh_sc(h_ids, r_ids, t_ids, ent_pair, rn_table):
    B = h_ids.shape[0]
    NC, NS, L = 2, 16, 16
    NW = NC * NS
    PW = B // NW
    C = min(128, PW)
    NCH = PW // C
    G = C // L

    mesh = plsc.VectorSubcoreMesh(
        core_axis_name="c", subcore_axis_name="s", num_cores=NC, num_subcores=NS)

    @functools.partial(
        pl.kernel,
        mesh=mesh,
        out_type=jax.ShapeDtypeStruct((B,), jnp.float32),
        compiler_params=pltpu.CompilerParams(
            needs_layout_passes=False, use_tc_tiling_on_sc=True),
        scratch_types=[
            pltpu.VMEM((2, C), jnp.int32),
            pltpu.VMEM((2, C), jnp.int32),
            pltpu.VMEM((2, C), jnp.int32),
            pltpu.VMEM((2, C), jnp.int32),
            pltpu.VMEM((2, C), jnp.int32),
            pltpu.VMEM((2, C, 2 * DIM), jnp.float32),
            pltpu.VMEM((2, C, 2 * DIM), jnp.float32),
            pltpu.VMEM((2, C, 2 * DIM), jnp.float32),
            pltpu.VMEM((DIM, L), jnp.float32),
            pltpu.VMEM((PW,), jnp.float32),
            pltpu.SemaphoreType.DMA,
            pltpu.SemaphoreType.DMA,
        ],
    )
    def _k(h_hbm, r_hbm, t_hbm, ent_hbm, rn_hbm, out_hbm,
           hids, tids, hrow, trow, rids, hrows, trows, rn_rows,
           u_scr, outv, sem0, sem1):
        wid = lax.axis_index("s") * NC + lax.axis_index("c")
        lane = lax.iota(jnp.int32, L)
        sems = (sem0, sem1)

        def fire(ch, s):
            base = wid * PW + ch * C
            pltpu.sync_copy(h_hbm.at[pl.ds(base, C)], hids.at[s])
            pltpu.sync_copy(t_hbm.at[pl.ds(base, C)], tids.at[s])
            pltpu.sync_copy(r_hbm.at[pl.ds(base, C)], rids.at[s])

            @pl.loop(0, G)
            def _halve(i):
                sl = pl.ds(i * L, L)
                hv = hids[s, sl]
                tv = tids[s, sl]
                hrow[s, sl] = ((hv >> (PAIR_SH + 1)) << PAIR_SH) + (hv & (PAIR_BN - 1))
                trow[s, sl] = ((tv >> (PAIR_SH + 1)) << PAIR_SH) + (tv & (PAIR_BN - 1))

            pltpu.async_copy(ent_hbm.at[hrow.at[s]], hrows.at[s], sems[s])
            pltpu.async_copy(ent_hbm.at[trow.at[s]], trows.at[s], sems[s])
            pltpu.async_copy(rn_hbm.at[rids.at[s]], rn_rows.at[s], sems[s])

        def drain(s):
            pltpu.make_async_copy(
                ent_hbm.at[pl.ds(0, C)], hrows.at[s], sems[s]).wait()
            pltpu.make_async_copy(
                ent_hbm.at[pl.ds(0, C)], trows.at[s], sems[s]).wait()
            pltpu.make_async_copy(
                rn_hbm.at[pl.ds(0, C)], rn_rows.at[s], sems[s]).wait()

        fire(0, 0)
        for ch in range(NCH):
            s = ch & 1
            if ch + 1 < NCH:
                fire(ch + 1, (ch + 1) & 1)
            drain(s)

            @pl.loop(0, G)
            def _group(g, ch=ch, s=s):
                row = g * L + lane
                sl = pl.ds(g * L, L)
                cdh = ((hids[s, sl] >> PAIR_SH) & 1) * DIM
                cdt = ((tids[s, sl] >> PAIR_SH) & 1) * DIM
                cdn = jnp.full((L,), DIM, jnp.int32)
                un = jnp.zeros((L,), jnp.float32)
                nn = jnp.zeros((L,), jnp.float32)
                for d in range(DIM):
                    hv = plsc.load_gather(hrows.at[s], [row, cdh])
                    tv = plsc.load_gather(trows.at[s], [row, cdt])
                    nv = plsc.load_gather(rn_rows.at[s], [row, cdn])
                    uv = hv - tv
                    u_scr[d] = uv
                    un = un + uv * nv
                    nn = nn + nv * nv
                    if d + 1 < DIM:
                        cdh = cdh + 1
                        cdt = cdt + 1
                        cdn = cdn + 1
                gamma = un / jnp.maximum(nn, 1e-24)
                cdr = jnp.zeros((L,), jnp.int32)
                cdn2 = jnp.full((L,), DIM, jnp.int32)
                acc = jnp.zeros((L,), jnp.float32)
                for d in range(DIM):
                    rv = plsc.load_gather(rn_rows.at[s], [row, cdr])
                    nv = plsc.load_gather(rn_rows.at[s], [row, cdn2])
                    acc = acc + jnp.abs(u_scr[d] + rv - gamma * nv)
                    if d + 1 < DIM:
                        cdr = cdr + 1
                        cdn2 = cdn2 + 1
                outv[pl.ds(ch * C + g * L, L)] = acc

        pltpu.sync_copy(outv, out_hbm.at[pl.ds(wid * PW, PW)])

    return _k(h_ids, r_ids, t_ids, ent_pair, rn_table)


def kernel(h_ids, r_ids, t_ids, entity_emb, relation_emb, normal_vec):
    ent_pair = _pair_rows_tc(entity_emb.T)
    rn_table = jnp.concatenate([relation_emb, normal_vec], axis=1)
    return _transh_sc(h_ids, r_ids, t_ids, ent_pair, rn_table)


# speedup vs baseline: 15.8036x; 1.0450x over previous
import functools

import jax
import jax.numpy as jnp
from jax import lax
from jax.experimental import pallas as pl
from jax.experimental.pallas import tpu as pltpu
from jax.experimental.pallas import tpu_sc as plsc

DIM = 64
PAIR_BN = 16384
PAIR_SH = 14


def _pair_rows_tc(ent_t):
    NE = ent_t.shape[1]
    BN = PAIR_BN
    grid = pl.cdiv(NE, 2 * BN)

    def body(in_ref, out_ref):
        x = in_ref[...]
        out_ref[...] = jnp.concatenate(
            [x[:, :BN].T, x[:, BN:].T], axis=1)

    return pl.pallas_call(
        body,
        grid=(grid,),
        in_specs=[pl.BlockSpec((DIM, 2 * BN), lambda j: (0, j))],
        out_specs=pl.BlockSpec((BN, 2 * DIM), lambda j: (j, 0)),
        out_shape=jax.ShapeDtypeStruct((grid * BN, 2 * DIM), jnp.float32),
        compiler_params=pltpu.CompilerParams(
            fuse_transposed_lhs_in_matmul=True),
    )(ent_t)


def _transh_sc(h_ids, r_ids, t_ids, ent_pair, rn_table):
    B = h_ids.shape[0]
    NC, NS, L = 2, 16, 16
    NW = NC * NS
    PW = B // NW
    C = min(128, PW)
    NCH = PW // C
    G = C // L

    mesh = plsc.VectorSubcoreMesh(
        core_axis_name="c", subcore_axis_name="s", num_cores=NC, num_subcores=NS)

    @functools.partial(
        pl.kernel,
        mesh=mesh,
        out_type=jax.ShapeDtypeStruct((B,), jnp.float32),
        compiler_params=pltpu.CompilerParams(
            needs_layout_passes=False, use_tc_tiling_on_sc=True),
        scratch_types=[
            pltpu.VMEM((2, C), jnp.int32),
            pltpu.VMEM((2, C), jnp.int32),
            pltpu.VMEM((2, C), jnp.int32),
            pltpu.VMEM((2, C), jnp.int32),
            pltpu.VMEM((2, C), jnp.int32),
            pltpu.VMEM((2, C, 2 * DIM), jnp.float32),
            pltpu.VMEM((2, C, 2 * DIM), jnp.float32),
            pltpu.VMEM((2, C, 2 * DIM), jnp.float32),
            pltpu.VMEM((DIM, L), jnp.float32),
            pltpu.VMEM((PW,), jnp.float32),
            pltpu.SemaphoreType.DMA,
            pltpu.SemaphoreType.DMA,
        ],
    )
    def _k(h_hbm, r_hbm, t_hbm, ent_hbm, rn_hbm, out_hbm,
           hids, tids, hrow, trow, rids, hrows, trows, rn_rows,
           u_scr, outv, sem0, sem1):
        wid = lax.axis_index("s") * NC + lax.axis_index("c")
        lane = lax.iota(jnp.int32, L)
        sems = (sem0, sem1)

        def fire(ch, s):
            base = wid * PW + ch * C
            pltpu.sync_copy(h_hbm.at[pl.ds(base, C)], hids.at[s])
            pltpu.sync_copy(t_hbm.at[pl.ds(base, C)], tids.at[s])
            pltpu.sync_copy(r_hbm.at[pl.ds(base, C)], rids.at[s])

            @pl.loop(0, G)
            def _halve(i):
                sl = pl.ds(i * L, L)
                hv = hids[s, sl]
                tv = tids[s, sl]
                hrow[s, sl] = ((hv >> (PAIR_SH + 1)) << PAIR_SH) + (hv & (PAIR_BN - 1))
                trow[s, sl] = ((tv >> (PAIR_SH + 1)) << PAIR_SH) + (tv & (PAIR_BN - 1))

            pltpu.async_copy(ent_hbm.at[hrow.at[s]], hrows.at[s], sems[s])
            pltpu.async_copy(ent_hbm.at[trow.at[s]], trows.at[s], sems[s])
            pltpu.async_copy(rn_hbm.at[rids.at[s]], rn_rows.at[s], sems[s])

        def drain(s):
            pltpu.make_async_copy(
                ent_hbm.at[pl.ds(0, C)], hrows.at[s], sems[s]).wait()
            pltpu.make_async_copy(
                ent_hbm.at[pl.ds(0, C)], trows.at[s], sems[s]).wait()
            pltpu.make_async_copy(
                rn_hbm.at[pl.ds(0, C)], rn_rows.at[s], sems[s]).wait()

        fire(0, 0)
        for ch in range(NCH):
            s = ch & 1
            if ch + 1 < NCH:
                fire(ch + 1, (ch + 1) & 1)
            drain(s)

            @pl.loop(0, G)
            def _group(g, ch=ch, s=s):
                row = g * L + lane
                sl = pl.ds(g * L, L)
                cdh = ((hids[s, sl] >> PAIR_SH) & 1) * DIM
                cdt = ((tids[s, sl] >> PAIR_SH) & 1) * DIM
                cdn = jnp.full((L,), DIM, jnp.int32)
                un = jnp.zeros((L,), jnp.float32)
                nn = jnp.zeros((L,), jnp.float32)
                for d in range(DIM):
                    hv = plsc.load_gather(hrows.at[s], [row, cdh])
                    tv = plsc.load_gather(trows.at[s], [row, cdt])
                    nv = plsc.load_gather(rn_rows.at[s], [row, cdn])
                    uv = hv - tv
                    u_scr[d] = uv
                    un = un + uv * nv
                    nn = nn + nv * nv
                    if d + 1 < DIM:
                        cdh = cdh + 1
                        cdt = cdt + 1
                        cdn = cdn + 1
                gamma = un / jnp.maximum(nn, 1e-24)
                cdr = jnp.zeros((L,), jnp.int32)
                cdn2 = jnp.full((L,), DIM, jnp.int32)
                acc = jnp.zeros((L,), jnp.float32)
                for d in range(DIM):
                    rv = plsc.load_gather(rn_rows.at[s], [row, cdr])
                    nv = plsc.load_gather(rn_rows.at[s], [row, cdn2])
                    acc = acc + jnp.abs(u_scr[d] + rv - gamma * nv)
                    if d + 1 < DIM:
                        cdr = cdr + 1
                        cdn2 = cdn2 + 1
                outv[pl.ds(ch * C + g * L, L)] = acc

        pltpu.sync_copy(outv, out_hbm.at[pl.ds(wid * PW, PW)])

    return _k(h_ids, r_ids, t_ids, ent_pair, rn_table)


def kernel(h_ids, r_ids, t_ids, entity_emb, relation_emb, normal_vec):
    ent_pair = _pair_rows_tc(entity_emb.T)
    rn_table = jnp.concatenate([relation_emb, normal_vec], axis=1)
    return _transh_sc(h_ids, r_ids, t_ids, ent_pair, rn_table)


# speedup vs baseline: 16.2329x; 1.0272x over previous
import functools

import jax
import jax.numpy as jnp
from jax import lax
from jax.experimental import pallas as pl
from jax.experimental.pallas import tpu as pltpu
from jax.experimental.pallas import tpu_sc as plsc

DIM = 64
PAIR_BN = 16384
PAIR_SH = 14


def _pair_rows_tc(ent_t):
    NE = ent_t.shape[1]
    BN = PAIR_BN
    grid = pl.cdiv(NE, 2 * BN)

    def body(in_ref, out_ref):
        x = in_ref[...]
        out_ref[...] = jnp.concatenate(
            [x[:, :BN].T, x[:, BN:].T], axis=1)

    return pl.pallas_call(
        body,
        grid=(grid,),
        in_specs=[pl.BlockSpec((DIM, 2 * BN), lambda j: (0, j))],
        out_specs=pl.BlockSpec((BN, 2 * DIM), lambda j: (j, 0)),
        out_shape=jax.ShapeDtypeStruct((grid * BN, 2 * DIM), jnp.float32),
    )(ent_t)


def _transh_sc(h_ids, r_ids, t_ids, ent_pair, rn_table):
    B = h_ids.shape[0]
    NC, NS, L = 2, 16, 16
    NW = NC * NS
    PW = B // NW
    C = min(128, PW)
    NCH = PW // C
    G = C // L

    mesh = plsc.VectorSubcoreMesh(
        core_axis_name="c", subcore_axis_name="s", num_cores=NC, num_subcores=NS)

    @functools.partial(
        pl.kernel,
        mesh=mesh,
        out_type=jax.ShapeDtypeStruct((B,), jnp.float32),
        compiler_params=pltpu.CompilerParams(
            needs_layout_passes=False, use_tc_tiling_on_sc=True),
        scratch_types=[
            pltpu.VMEM((2, C), jnp.int32),
            pltpu.VMEM((2, C), jnp.int32),
            pltpu.VMEM((2, C), jnp.int32),
            pltpu.VMEM((2, C), jnp.int32),
            pltpu.VMEM((2, C), jnp.int32),
            pltpu.VMEM((2, C, 2 * DIM), jnp.float32),
            pltpu.VMEM((2, C, 2 * DIM), jnp.float32),
            pltpu.VMEM((2, C, 2 * DIM), jnp.float32),
            pltpu.VMEM((DIM, L), jnp.float32),
            pltpu.VMEM((DIM, L), jnp.float32),
            pltpu.VMEM((PW,), jnp.float32),
            pltpu.SemaphoreType.DMA,
            pltpu.SemaphoreType.DMA,
        ],
    )
    def _k(h_hbm, r_hbm, t_hbm, ent_hbm, rn_hbm, out_hbm,
           hids, tids, hrow, trow, rids, hrows, trows, rn_rows,
           u_scr, n_scr, outv, sem0, sem1):
        wid = lax.axis_index("s") * NC + lax.axis_index("c")
        lane = lax.iota(jnp.int32, L)
        sems = (sem0, sem1)

        def fire(ch, s):
            base = wid * PW + ch * C
            pltpu.sync_copy(h_hbm.at[pl.ds(base, C)], hids.at[s])
            pltpu.sync_copy(t_hbm.at[pl.ds(base, C)], tids.at[s])
            pltpu.sync_copy(r_hbm.at[pl.ds(base, C)], rids.at[s])

            @pl.loop(0, G)
            def _halve(i):
                sl = pl.ds(i * L, L)
                hv = hids[s, sl]
                tv = tids[s, sl]
                hrow[s, sl] = ((hv >> (PAIR_SH + 1)) << PAIR_SH) + (hv & (PAIR_BN - 1))
                trow[s, sl] = ((tv >> (PAIR_SH + 1)) << PAIR_SH) + (tv & (PAIR_BN - 1))

            pltpu.async_copy(ent_hbm.at[hrow.at[s]], hrows.at[s], sems[s])
            pltpu.async_copy(ent_hbm.at[trow.at[s]], trows.at[s], sems[s])
            pltpu.async_copy(rn_hbm.at[rids.at[s]], rn_rows.at[s], sems[s])

        def drain(s):
            pltpu.make_async_copy(
                ent_hbm.at[pl.ds(0, C)], hrows.at[s], sems[s]).wait()
            pltpu.make_async_copy(
                ent_hbm.at[pl.ds(0, C)], trows.at[s], sems[s]).wait()
            pltpu.make_async_copy(
                rn_hbm.at[pl.ds(0, C)], rn_rows.at[s], sems[s]).wait()

        fire(0, 0)
        for ch in range(NCH):
            s = ch & 1
            if ch + 1 < NCH:
                fire(ch + 1, (ch + 1) & 1)
            drain(s)

            @pl.loop(0, G)
            def _group(g, ch=ch, s=s):
                row = g * L + lane
                sl = pl.ds(g * L, L)
                cdh = ((hids[s, sl] >> PAIR_SH) & 1) * DIM
                cdt = ((tids[s, sl] >> PAIR_SH) & 1) * DIM
                cdn = jnp.full((L,), DIM, jnp.int32)
                un = jnp.zeros((L,), jnp.float32)
                nn = jnp.zeros((L,), jnp.float32)
                for d in range(DIM):
                    hv = plsc.load_gather(hrows.at[s], [row, cdh])
                    tv = plsc.load_gather(trows.at[s], [row, cdt])
                    nv = plsc.load_gather(rn_rows.at[s], [row, cdn])
                    uv = hv - tv
                    u_scr[d] = uv
                    n_scr[d] = nv
                    un = un + uv * nv
                    nn = nn + nv * nv
                    if d + 1 < DIM:
                        cdh = cdh + 1
                        cdt = cdt + 1
                        cdn = cdn + 1
                gamma = un / jnp.maximum(nn, 1e-24)
                cdr = jnp.zeros((L,), jnp.int32)
                acc = jnp.zeros((L,), jnp.float32)
                for d in range(DIM):
                    rv = plsc.load_gather(rn_rows.at[s], [row, cdr])
                    acc = acc + jnp.abs(u_scr[d] + rv - gamma * n_scr[d])
                    if d + 1 < DIM:
                        cdr = cdr + 1
                outv[pl.ds(ch * C + g * L, L)] = acc

        pltpu.sync_copy(outv, out_hbm.at[pl.ds(wid * PW, PW)])

    return _k(h_ids, r_ids, t_ids, ent_pair, rn_table)


def kernel(h_ids, r_ids, t_ids, entity_emb, relation_emb, normal_vec):
    ent_pair = _pair_rows_tc(entity_emb.T)
    rn_table = jnp.concatenate([relation_emb, normal_vec], axis=1)
    return _transh_sc(h_ids, r_ids, t_ids, ent_pair, rn_table)
